# baseline (speedup 1.0000x reference)
import functools

import jax
import jax.numpy as jnp
from jax import lax
from jax.experimental import pallas as pl
from jax.experimental.pallas import tpu as pltpu
from jax.experimental.pallas import tpu_sc as plsc

NC = 2
NS = 16
LANES = 16
CHUNK = 128


def _pad_nodes(n):
  return ((n + NS * CHUNK - 1) // (NS * CHUNK)) * (NS * CHUNK)


def _worker_perm(e):
  import numpy as np
  n_chunks = e // CHUNK
  nw = NC * NS
  rounds = (n_chunks + nw - 1) // nw
  rounds = ((rounds + 3) // 4) * 4
  ids = []
  for w in range(nw):
    lst = list(range(w, n_chunks, nw))
    lst += [n_chunks] * (rounds - len(lst))
    ids += lst
  return np.asarray(ids, dtype=np.int32), rounds


def _sc_degree(cols3d, n_nodes, rounds):
  n_pad = _pad_nodes(n_nodes)
  rows_per_sub = n_pad // NS
  mesh = plsc.VectorSubcoreMesh(core_axis_name="c", subcore_axis_name="s", num_cores=NC, num_subcores=NS)

  @functools.partial(
      pl.kernel,
      out_type=jax.ShapeDtypeStruct((NC, n_pad, LANES), jnp.float32),
      mesh=mesh,
      scratch_types=[
          pltpu.VMEM((rounds, CHUNK), jnp.int32),
          pltpu.VMEM((CHUNK, LANES), jnp.float32),
          pltpu.VMEM((rows_per_sub, LANES), jnp.float32),
          pltpu.VMEM_SHARED((n_pad, LANES), jnp.float32),
      ],
      compiler_params=pltpu.CompilerParams(use_tc_tiling_on_sc=False),
  )
  def deg_kernel(cols_hbm, out_hbm, idx_v, ones_v, zero_v, acc_sh):
    c = lax.axis_index("c")
    s = lax.axis_index("s")
    wid = s * NC + c

    zero16 = jnp.zeros((LANES,), jnp.float32)
    one_row = jnp.where(lax.iota(jnp.int32, LANES) == 0, 1.0, 0.0)

    @pl.loop(0, rows_per_sub)
    def _(r):
      zero_v[r, :] = zero16

    @pl.loop(0, CHUNK)
    def _(r):
      ones_v[r, :] = one_row

    pltpu.sync_copy(zero_v, acc_sh.at[pl.ds(s * rows_per_sub, rows_per_sub)])
    pltpu.sync_copy(cols_hbm.at[wid], idx_v)
    plsc.subcore_barrier()

    @pl.loop(0, rounds)
    def _(k):
      pltpu.sync_copy(ones_v, acc_sh.at[idx_v.at[k]], add=True)

    plsc.subcore_barrier()
    pltpu.sync_copy(
        acc_sh.at[pl.ds(s * rows_per_sub, rows_per_sub)],
        out_hbm.at[c, pl.ds(s * rows_per_sub, rows_per_sub)],
    )

  return deg_kernel(cols3d)


def _sc_spmm(h2, src3d, dst3d, n_nodes, rounds):
  d = h2.shape[1]
  n_pad = _pad_nodes(n_nodes)
  rows_per_sub = n_pad // NS
  zrows = CHUNK
  mesh = plsc.VectorSubcoreMesh(core_axis_name="c", subcore_axis_name="s", num_cores=NC, num_subcores=NS)

  @functools.partial(
      pl.kernel,
      out_type=jax.ShapeDtypeStruct((NC, n_pad, d), jnp.float32),
      mesh=mesh,
      scratch_types=[
          pltpu.VMEM((rounds // 2, CHUNK), jnp.int32),
          pltpu.VMEM((rounds // 2, CHUNK), jnp.int32),
          pltpu.VMEM((CHUNK, d), jnp.float32),
          pltpu.VMEM((CHUNK, d), jnp.float32),
          pltpu.VMEM_SHARED((n_pad, d), jnp.float32),
          pltpu.SemaphoreType.DMA,
          pltpu.SemaphoreType.DMA,
      ],
  )
  def spmm_kernel(h2_hbm, src_hbm, dst_hbm, out_hbm, sidx, didx, gbuf0, gbuf1,
                  acc_sh, sem0, sem1):
    c = lax.axis_index("c")
    s = lax.axis_index("s")
    wid = s * NC + c

    zero16 = jnp.zeros((LANES,), jnp.float32)

    @pl.loop(0, zrows)
    def _(r):
      @pl.loop(0, d, step=LANES)
      def _(j):
        gbuf0[r, pl.ds(j, LANES)] = zero16

    @pl.loop(0, rows_per_sub, step=zrows)
    def _(r0):
      pltpu.sync_copy(
          gbuf0.at[pl.ds(0, zrows)],
          acc_sh.at[pl.ds(s * rows_per_sub + r0, zrows)],
      )

    plsc.subcore_barrier()

    half = rounds // 2
    for ph in range(2):
      pltpu.sync_copy(src_hbm.at[wid, pl.ds(ph * half, half)], sidx)
      pltpu.sync_copy(dst_hbm.at[wid, pl.ds(ph * half, half)], didx)
      pltpu.make_async_copy(h2_hbm.at[sidx.at[0]], gbuf0, sem0).start()

      @pl.loop(0, half, step=2)
      def _(k):
        pltpu.make_async_copy(h2_hbm.at[sidx.at[k]], gbuf0, sem0).wait()
        pltpu.make_async_copy(h2_hbm.at[sidx.at[k + 1]], gbuf1, sem1).start()
        pltpu.sync_copy(gbuf0, acc_sh.at[didx.at[k]], add=True)
        pltpu.make_async_copy(h2_hbm.at[sidx.at[k + 1]], gbuf1, sem1).wait()

        @pl.when(k + 2 < half)
        def _():
          pltpu.make_async_copy(h2_hbm.at[sidx.at[k + 2]], gbuf0, sem0).start()

        pltpu.sync_copy(gbuf1, acc_sh.at[didx.at[k + 1]], add=True)

    plsc.subcore_barrier()
    pltpu.sync_copy(
        acc_sh.at[pl.ds(s * rows_per_sub, rows_per_sub)],
        out_hbm.at[c, pl.ds(s * rows_per_sub, rows_per_sub)],
    )

  return spmm_kernel(h2, src3d, dst3d)


def _tc_linear(x, w):
  n, d_in = x.shape
  d_out = w.shape[0]
  bm = 1000

  def body(x_ref, w_ref, o_ref):
    o_ref[...] = lax.dot_general(
        x_ref[...], w_ref[...],
        (((1,), (1,)), ((), ())),
        precision=lax.Precision.HIGHEST,
    )

  return pl.pallas_call(
      body,
      grid=(n // bm,),
      in_specs=[
          pl.BlockSpec((bm, d_in), lambda i: (i, 0)),
          pl.BlockSpec((d_out, d_in), lambda i: (0, 0)),
      ],
      out_specs=pl.BlockSpec((bm, d_out), lambda i: (i, 0)),
      out_shape=jax.ShapeDtypeStruct((n, d_out), jnp.float32),
  )(x, w)


def _tc_scale(h, degp):
  n, d = h.shape
  bm = 1000

  def body(h_ref, dp_ref, o_ref):
    deg = 1.0 + dp_ref[0, :, 0] + dp_ref[1, :, 0]
    o_ref[...] = h_ref[...] * lax.rsqrt(deg)[:, None]

  return pl.pallas_call(
      body,
      grid=(n // bm,),
      in_specs=[
          pl.BlockSpec((bm, d), lambda i: (i, 0)),
          pl.BlockSpec((NC, bm, LANES), lambda i: (0, i, 0)),
      ],
      out_specs=pl.BlockSpec((bm, d), lambda i: (i, 0)),
      out_shape=jax.ShapeDtypeStruct((n, d), jnp.float32),
  )(h, degp)


def _tc_combine(q, degp, h2):
  n, d = h2.shape
  bm = 1000

  def body(q_ref, dp_ref, h2_ref, o_ref):
    deg = 1.0 + dp_ref[0, :, 0] + dp_ref[1, :, 0]
    agg = q_ref[0] + q_ref[1] + h2_ref[...]
    o_ref[...] = jnp.maximum(agg * lax.rsqrt(deg)[:, None], 0.0)

  return pl.pallas_call(
      body,
      grid=(n // bm,),
      in_specs=[
          pl.BlockSpec((NC, bm, d), lambda i: (0, i, 0)),
          pl.BlockSpec((NC, bm, LANES), lambda i: (0, i, 0)),
          pl.BlockSpec((bm, d), lambda i: (i, 0)),
      ],
      out_specs=pl.BlockSpec((bm, d), lambda i: (i, 0)),
      out_shape=jax.ShapeDtypeStruct((n, d), jnp.float32),
  )(q, degp, h2)


def kernel(x, edge_index, W):
  n = x.shape[0]
  n_pad = _pad_nodes(n)
  garbage_row = n_pad - 8
  ei = edge_index.astype(jnp.int32)
  e = ei.shape[1]
  n_chunks = e // CHUNK
  nw = NC * NS

  perm, rounds = _worker_perm(e)
  dst2d = ei[0].reshape(n_chunks, CHUNK)
  src2d = ei[1].reshape(n_chunks, CHUNK)
  pad_src = jnp.zeros((1, CHUNK), jnp.int32)
  pad_garbage = jnp.full((1, CHUNK), garbage_row, jnp.int32)
  src3d = jnp.concatenate([src2d, pad_src])[perm].reshape(nw, rounds, CHUNK)
  dst3d = jnp.concatenate([dst2d, pad_garbage])[perm].reshape(nw, rounds, CHUNK)
  cols3d = jnp.concatenate([src2d, pad_garbage])[perm].reshape(nw, rounds, CHUNK)

  degp = _sc_degree(cols3d, n, rounds)[:, :n]
  h = _tc_linear(x, W)
  h2 = _tc_scale(h, degp)
  q = _sc_spmm(h2, src3d, dst3d, n, rounds)[:, :n]
  return _tc_combine(q, degp, h2)
